# baseline (speedup 1.0000x reference)
import jax
import jax.numpy as jnp
from jax.experimental import pallas as pl

N = 10000
E = 320000
H0 = 128
OUT = 256
LAT = 64
NT = 10
NB = 55
SH = 16
RMAX = 5.0

BE = 1280
BN = 1000


def _edge_body(oh_ref, sh_ref, h0_ref, bt_ref, el_ref,
               wla_ref, wlb_ref, wb_ref, wp_ref, be_ref, me_ref,
               lat_ref, ef_ref, co_ref):
    lat = jnp.dot(oh_ref[...], wla_ref[...], preferred_element_type=jnp.float32)
    lat = lat + jnp.dot(sh_ref[...], wlb_ref[...], preferred_element_type=jnp.float32)
    lat_ref[...] = lat
    bt = bt_ref[...]
    sel = (bt == jax.lax.broadcasted_iota(jnp.int32, (BE, NB), 1)
           ).astype(jnp.float32)
    mrow = jnp.dot(sel, me_ref[...], preferred_element_type=jnp.float32)
    src = h0_ref[...] * mrow
    ef = jnp.dot(lat, wb_ref[...], preferred_element_type=jnp.float32)
    ef = ef + jnp.dot(src, wp_ref[...], preferred_element_type=jnp.float32)
    ef_ref[...] = ef + be_ref[...]
    x = jnp.clip(el_ref[...] * (1.0 / RMAX), 0.0, 1.0)
    co_ref[...] = 0.5 * (jnp.cos(jnp.pi * x) + 1.0)


def _node_body(h0_ref, at_ref, emb_ref, mn_ref, wn_ref, bn_ref, nf_ref):
    at = at_ref[...]
    sel = (at == jax.lax.broadcasted_iota(jnp.int32, (BN, NT), 1)
           ).astype(jnp.float32)
    base = jnp.dot(sel, emb_ref[...], preferred_element_type=jnp.float32)
    mrow = jnp.dot(sel, mn_ref[...], preferred_element_type=jnp.float32)
    src = h0_ref[...] * mrow
    nf = base + jnp.dot(src, wn_ref[...], preferred_element_type=jnp.float32)
    nf_ref[...] = nf + bn_ref[...]


def kernel(node_h0, edge_h0, edge_index, atom_type, bond_type, edge_sh,
           edge_length, edge_one_hot, W_latent, W_edge_base, atom_embed,
           W_node_proj, b_node, W_edge_proj, b_edge, mask_nrme, mask_erme):
    bt2 = bond_type.reshape(E, 1)
    el2 = edge_length.reshape(E, 1)
    at2 = atom_type.reshape(N, 1)
    wla = W_latent[:NT]
    wlb = W_latent[NT:]
    be2 = b_edge.reshape(1, OUT)
    bn2 = b_node.reshape(1, OUT)

    n_eblk = E // BE
    row = lambda i: (i, 0)
    full = lambda i: (0, 0)

    lat, ef, co = pl.pallas_call(
        _edge_body,
        grid=(n_eblk,),
        in_specs=[
            pl.BlockSpec((BE, NT), row),
            pl.BlockSpec((BE, SH), row),
            pl.BlockSpec((BE, H0), row),
            pl.BlockSpec((BE, 1), row),
            pl.BlockSpec((BE, 1), row),
            pl.BlockSpec((NT, LAT), full),
            pl.BlockSpec((SH, LAT), full),
            pl.BlockSpec((LAT, OUT), full),
            pl.BlockSpec((H0, OUT), full),
            pl.BlockSpec((1, OUT), full),
            pl.BlockSpec((NB, H0), full),
        ],
        out_specs=[
            pl.BlockSpec((BE, LAT), row),
            pl.BlockSpec((BE, OUT), row),
            pl.BlockSpec((BE, 1), row),
        ],
        out_shape=[
            jax.ShapeDtypeStruct((E, LAT), jnp.float32),
            jax.ShapeDtypeStruct((E, OUT), jnp.float32),
            jax.ShapeDtypeStruct((E, 1), jnp.float32),
        ],
    )(edge_one_hot, edge_sh, edge_h0, bt2, el2,
      wla, wlb, W_edge_base, W_edge_proj, be2, mask_erme)

    nf = pl.pallas_call(
        _node_body,
        grid=(N // BN,),
        in_specs=[
            pl.BlockSpec((BN, H0), row),
            pl.BlockSpec((BN, 1), row),
            pl.BlockSpec((NT, OUT), full),
            pl.BlockSpec((NT, H0), full),
            pl.BlockSpec((H0, OUT), full),
            pl.BlockSpec((1, OUT), full),
        ],
        out_specs=pl.BlockSpec((BN, OUT), row),
        out_shape=jax.ShapeDtypeStruct((N, OUT), jnp.float32),
    )(node_h0, at2, atom_embed, mask_nrme, W_node_proj, bn2)

    active_edges = jnp.arange(E, dtype=jnp.int32)
    return (lat, nf, ef, co.reshape(E), active_edges)


# speedup vs baseline: 1.6141x; 1.6141x over previous
import jax
import jax.numpy as jnp
from jax.experimental import pallas as pl

N = 10000
E = 320000
H0 = 128
OUT = 256
LAT = 64
NT = 10
NB = 55
SH = 16
RMAX = 5.0

BE = 1280
BN = 1000


def _edge_body(oh_ref, sh_ref, h0_ref, bt_ref,
               wla_ref, wlb_ref, wb_ref, wp_ref, be_ref, me_ref,
               lat_ref, ef_ref):
    lat = jnp.dot(oh_ref[...], wla_ref[...], preferred_element_type=jnp.float32)
    lat = lat + jnp.dot(sh_ref[...], wlb_ref[...], preferred_element_type=jnp.float32)
    lat_ref[...] = lat
    bt = bt_ref[...]
    sel = (bt == jax.lax.broadcasted_iota(jnp.int32, (BE, NB), 1)
           ).astype(jnp.float32)
    mrow = jnp.dot(sel, me_ref[...], preferred_element_type=jnp.float32)
    src = h0_ref[...] * mrow
    ef = jnp.dot(lat, wb_ref[...], preferred_element_type=jnp.float32)
    ef = ef + jnp.dot(src, wp_ref[...], preferred_element_type=jnp.float32)
    ef_ref[...] = ef + be_ref[...]


def _cutoff_body(el_ref, co_ref, ae_ref):
    x = jnp.clip(el_ref[...] * (1.0 / RMAX), 0.0, 1.0)
    co_ref[...] = 0.5 * (jnp.cos(jnp.pi * x) + 1.0)
    rows, cols = ae_ref.shape
    ae_ref[...] = (jax.lax.broadcasted_iota(jnp.int32, (rows, cols), 0) * cols
                   + jax.lax.broadcasted_iota(jnp.int32, (rows, cols), 1))


def _node_body(h0_ref, at_ref, emb_ref, mn_ref, wn_ref, bn_ref, nf_ref):
    at = at_ref[...]
    sel = (at == jax.lax.broadcasted_iota(jnp.int32, (BN, NT), 1)
           ).astype(jnp.float32)
    base = jnp.dot(sel, emb_ref[...], preferred_element_type=jnp.float32)
    mrow = jnp.dot(sel, mn_ref[...], preferred_element_type=jnp.float32)
    src = h0_ref[...] * mrow
    nf = base + jnp.dot(src, wn_ref[...], preferred_element_type=jnp.float32)
    nf_ref[...] = nf + bn_ref[...]


def kernel(node_h0, edge_h0, edge_index, atom_type, bond_type, edge_sh,
           edge_length, edge_one_hot, W_latent, W_edge_base, atom_embed,
           W_node_proj, b_node, W_edge_proj, b_edge, mask_nrme, mask_erme):
    bt2 = bond_type.reshape(E, 1)
    el2 = edge_length.reshape(E // 128, 128)
    at2 = atom_type.reshape(N, 1)
    wla = W_latent[:NT]
    wlb = W_latent[NT:]
    be2 = b_edge.reshape(1, OUT)
    bn2 = b_node.reshape(1, OUT)

    n_eblk = E // BE
    row = lambda i: (i, 0)
    full = lambda i: (0, 0)

    lat, ef = pl.pallas_call(
        _edge_body,
        grid=(n_eblk,),
        in_specs=[
            pl.BlockSpec((BE, NT), row),
            pl.BlockSpec((BE, SH), row),
            pl.BlockSpec((BE, H0), row),
            pl.BlockSpec((BE, 1), row),
            pl.BlockSpec((NT, LAT), full),
            pl.BlockSpec((SH, LAT), full),
            pl.BlockSpec((LAT, OUT), full),
            pl.BlockSpec((H0, OUT), full),
            pl.BlockSpec((1, OUT), full),
            pl.BlockSpec((NB, H0), full),
        ],
        out_specs=[
            pl.BlockSpec((BE, LAT), row),
            pl.BlockSpec((BE, OUT), row),
        ],
        out_shape=[
            jax.ShapeDtypeStruct((E, LAT), jnp.float32),
            jax.ShapeDtypeStruct((E, OUT), jnp.float32),
        ],
    )(edge_one_hot, edge_sh, edge_h0, bt2,
      wla, wlb, W_edge_base, W_edge_proj, be2, mask_erme)

    co, ae = pl.pallas_call(
        _cutoff_body,
        grid=(1,),
        in_specs=[pl.BlockSpec((E // 128, 128), full)],
        out_specs=[
            pl.BlockSpec((E // 128, 128), full),
            pl.BlockSpec((E // 128, 128), full),
        ],
        out_shape=[
            jax.ShapeDtypeStruct((E // 128, 128), jnp.float32),
            jax.ShapeDtypeStruct((E // 128, 128), jnp.int32),
        ],
    )(el2)

    nf = pl.pallas_call(
        _node_body,
        grid=(N // BN,),
        in_specs=[
            pl.BlockSpec((BN, H0), row),
            pl.BlockSpec((BN, 1), row),
            pl.BlockSpec((NT, OUT), full),
            pl.BlockSpec((NT, H0), full),
            pl.BlockSpec((H0, OUT), full),
            pl.BlockSpec((1, OUT), full),
        ],
        out_specs=pl.BlockSpec((BN, OUT), row),
        out_shape=jax.ShapeDtypeStruct((N, OUT), jnp.float32),
    )(node_h0, at2, atom_embed, mask_nrme, W_node_proj, bn2)

    return (lat, nf, ef, co.reshape(E), ae.reshape(E))


# speedup vs baseline: 2.6145x; 1.6197x over previous
import jax
import jax.numpy as jnp
from jax.experimental import pallas as pl

N = 10000
E = 320000
H0 = 128
OUT = 256
LAT = 64
NT = 10
NB = 55
SH = 16
RMAX = 5.0
XW = NT + SH + 1

BE = 2560
BN = 2000


def _edge_body(x_ref, h0_ref, wl_ref, wc_ref, wp_ref, be_ref, me_ref,
               lat_ref, ef_ref):
    x = x_ref[...]
    lat_ref[...] = jnp.dot(x, wl_ref[...], preferred_element_type=jnp.float32)
    btf = x[:, XW - 1:XW]
    sel = (btf == jax.lax.broadcasted_iota(jnp.float32, (BE, NB), 1)
           ).astype(jnp.float32)
    mrow = jnp.dot(sel, me_ref[...], preferred_element_type=jnp.float32)
    src = h0_ref[...] * mrow
    ef = jnp.dot(x, wc_ref[...], preferred_element_type=jnp.float32)
    ef = ef + jnp.dot(src, wp_ref[...], preferred_element_type=jnp.float32)
    ef_ref[...] = ef + be_ref[...]


def _node_body(xn_ref, h0_ref, emb_ref, mn_ref, wn_ref, bn_ref, nf_ref):
    atf = xn_ref[...][:, 0:1]
    sel = (atf == jax.lax.broadcasted_iota(jnp.float32, (BN, NT), 1)
           ).astype(jnp.float32)
    base = jnp.dot(sel, emb_ref[...], preferred_element_type=jnp.float32)
    mrow = jnp.dot(sel, mn_ref[...], preferred_element_type=jnp.float32)
    src = h0_ref[...] * mrow
    nf = base + jnp.dot(src, wn_ref[...], preferred_element_type=jnp.float32)
    nf_ref[...] = nf + bn_ref[...]


def _cutoff_body(el_ref, co_ref, ae_ref):
    x = jnp.clip(el_ref[...] * (1.0 / RMAX), 0.0, 1.0)
    co_ref[...] = 0.5 * (jnp.cos(jnp.pi * x) + 1.0)
    rows, cols = ae_ref.shape
    ae_ref[...] = (jax.lax.broadcasted_iota(jnp.int32, (rows, cols), 0) * cols
                   + jax.lax.broadcasted_iota(jnp.int32, (rows, cols), 1))


def kernel(node_h0, edge_h0, edge_index, atom_type, bond_type, edge_sh,
           edge_length, edge_one_hot, W_latent, W_edge_base, atom_embed,
           W_node_proj, b_node, W_edge_proj, b_edge, mask_nrme, mask_erme):
    x = jnp.concatenate(
        [edge_one_hot, edge_sh, bond_type.astype(jnp.float32)[:, None]],
        axis=1)
    xn = atom_type.astype(jnp.float32)[:, None]
    el2 = edge_length.reshape(E // 128, 128)
    wl = jnp.concatenate([W_latent, jnp.zeros((1, LAT), jnp.float32)], axis=0)
    wc = wl @ W_edge_base
    be2 = b_edge.reshape(1, OUT)
    bn2 = b_node.reshape(1, OUT)

    row = lambda i: (i, 0)
    full = lambda i: (0, 0)

    lat, ef = pl.pallas_call(
        _edge_body,
        grid=(E // BE,),
        in_specs=[
            pl.BlockSpec((BE, XW), row),
            pl.BlockSpec((BE, H0), row),
            pl.BlockSpec((XW, LAT), full),
            pl.BlockSpec((XW, OUT), full),
            pl.BlockSpec((H0, OUT), full),
            pl.BlockSpec((1, OUT), full),
            pl.BlockSpec((NB, H0), full),
        ],
        out_specs=[
            pl.BlockSpec((BE, LAT), row),
            pl.BlockSpec((BE, OUT), row),
        ],
        out_shape=[
            jax.ShapeDtypeStruct((E, LAT), jnp.float32),
            jax.ShapeDtypeStruct((E, OUT), jnp.float32),
        ],
    )(x, edge_h0, wl, wc, W_edge_proj, be2, mask_erme)

    co, ae = pl.pallas_call(
        _cutoff_body,
        grid=(1,),
        in_specs=[pl.BlockSpec((E // 128, 128), full)],
        out_specs=[
            pl.BlockSpec((E // 128, 128), full),
            pl.BlockSpec((E // 128, 128), full),
        ],
        out_shape=[
            jax.ShapeDtypeStruct((E // 128, 128), jnp.float32),
            jax.ShapeDtypeStruct((E // 128, 128), jnp.int32),
        ],
    )(el2)

    nf = pl.pallas_call(
        _node_body,
        grid=(N // BN,),
        in_specs=[
            pl.BlockSpec((BN, 1), row),
            pl.BlockSpec((BN, H0), row),
            pl.BlockSpec((NT, OUT), full),
            pl.BlockSpec((NT, H0), full),
            pl.BlockSpec((H0, OUT), full),
            pl.BlockSpec((1, OUT), full),
        ],
        out_specs=pl.BlockSpec((BN, OUT), row),
        out_shape=jax.ShapeDtypeStruct((N, OUT), jnp.float32),
    )(xn, node_h0, atom_embed, mask_nrme, W_node_proj, bn2)

    return (lat, nf, ef, co.reshape(E), ae.reshape(E))
